# baseline (speedup 1.0000x reference)
import functools

import jax
import jax.numpy as jnp
from jax import lax
from jax.experimental import pallas as pl
from jax.experimental.pallas import tpu as pltpu
from jax.experimental.pallas import tpu_sc as plsc

R = 1000
B = 1024
N = 10000
NPARAM = 128

def _sc_gather(tables, rel_ids):
    info = plsc.get_sparse_core_info()
    nc, ns = info.num_cores, info.num_subcores
    nw = nc * ns
    b_per_w = B // nw
    mesh = plsc.VectorSubcoreMesh(core_axis_name="c", subcore_axis_name="s")

    packed = jnp.pad(tables[0].T, ((0, NPARAM - 1), (0, 0)))
    for j in range(1, 13):
        packed = packed + jnp.pad(tables[j].T, ((j, NPARAM - 1 - j), (0, 0)))
    table = packed.T

    @functools.partial(
        pl.kernel,
        mesh=mesh,
        out_type=jax.ShapeDtypeStruct((B, NPARAM), jnp.float32),
        scratch_types=[
            pltpu.VMEM((b_per_w,), jnp.int32),
            pltpu.VMEM((b_per_w, NPARAM), jnp.float32),
            pltpu.SemaphoreType.DMA,
        ],
    )
    def k(table_hbm, idx_hbm, out_hbm, idx_v, rows_v, sem):
        wid = lax.axis_index("s") * nc + lax.axis_index("c")
        base = wid * b_per_w
        pltpu.sync_copy(idx_hbm.at[pl.ds(base, b_per_w)], idx_v)
        pltpu.async_copy(table_hbm.at[idx_v], rows_v, sem).wait()
        pltpu.sync_copy(rows_v, out_hbm.at[pl.ds(base, b_per_w)])

    return k(table, rel_ids)


CB = 128
NCB = 8
CH = 1000


def _softplus(x):
    return jnp.maximum(x, 0.0) + jnp.log1p(jnp.exp(-jnp.abs(x)))


def _main_body(gam_ref, p_ref, base_ref,
               ssr_ref, dsr_ref, fsr_ref, sso_ref, dso_ref, fso_ref,
               sro_ref, dro_ref, fro_ref, logits_ref, hb_ref,
               csr_ref, cso_ref, cro_ref, macc_ref):
    c = pl.program_id(0)
    k = pl.program_id(1)
    par = lax.rem(c, 2)
    qar = lax.rem(c + 1, 2)

    @pl.when(c < NCB)
    def _():
        for t, (f_ref, c_ref) in enumerate(((fsr_ref, csr_ref),
                                            (fso_ref, cso_ref),
                                            (fro_ref, cro_ref))):
            x = f_ref[...]
            c_ref[pl.ds(par * N + k * CH, CH), :] = x
            mx = jnp.broadcast_to(jnp.max(x, axis=0, keepdims=True), (8, CB))
            row0 = (par * 3 + t) * 8

            @pl.when(k == 0)
            def _():
                macc_ref[pl.ds(row0, 8), :] = mx

            @pl.when(k > 0)
            def _():
                macc_ref[pl.ds(row0, 8), :] = jnp.maximum(
                    macc_ref[pl.ds(row0, 8), :], mx)

    @pl.when(c > 0)
    def _():
        def row(j):
            return p_ref[j:j + 1, :]

        def inv_norm(t, wfreq):
            row0 = (qar * 3 + t) * 8
            m = jnp.log1p(jnp.maximum(
                jnp.max(macc_ref[pl.ds(row0, 8), :], axis=0, keepdims=True),
                0.0))
            return wfreq / (m + 1e-8)

        def branch(seen, dt, c_ref, lam, wrec, inv, bias, wstale=None):
            freq = c_ref[pl.ds(qar * N + k * CH, CH), :]
            dtf = jnp.log(1.0 + jnp.maximum(dt, 0.0))
            rec = jnp.exp(-lam * dtf) * seen
            ff = jnp.log(1.0 + jnp.maximum(freq, 0.0))
            score = wrec * rec + ff * inv * seen + bias
            if wstale is not None:
                score = score - wstale * (seen - rec * seen)
            return jnp.tanh(score) * seen

        g_sr = branch(ssr_ref[...], dsr_ref[...], csr_ref,
                      _softplus(row(0)), row(1), inv_norm(0, row(2)),
                      row(4), wstale=row(3))
        g_so = branch(sso_ref[...], dso_ref[...], cso_ref,
                      _softplus(row(5)), row(6), inv_norm(1, row(7)),
                      row(8))
        g_ro = branch(sro_ref[...], dro_ref[...], cro_ref,
                      _softplus(row(9)), row(10), inv_norm(2, row(11)),
                      row(12))

        hb = gam_ref[0] * g_sr + (gam_ref[1] * 0.5) * (g_so + g_ro)
        hb_ref[...] = hb
        logits_ref[...] = base_ref[...] + hb


def _main(gammas, params_t, base, ssr, dsr, fsr,
          sso, dso, fso, sro, dro, fro):
    other = pl.BlockSpec((CH, CB), lambda c, k: (k, jnp.maximum(c - 1, 0)))
    freq = pl.BlockSpec((CH, CB), lambda c, k: (k, jnp.minimum(c, NCB - 1)))
    specs = [
        pl.BlockSpec(memory_space=pltpu.SMEM),
        pl.BlockSpec((NPARAM, CB),
                     lambda c, k: (0, jnp.maximum(c - 1, 0))),
        other,
        other, other, freq,
        other, other, freq,
        other, other, freq,
    ]
    return pl.pallas_call(
        _main_body,
        grid=(NCB + 1, N // CH),
        in_specs=specs,
        out_specs=[other, other],
        out_shape=[jax.ShapeDtypeStruct((N, B), jnp.float32)] * 2,
        scratch_shapes=[pltpu.VMEM((2 * N, CB), jnp.float32)] * 3
        + [pltpu.VMEM((48, CB), jnp.float32)],
        compiler_params=pltpu.CompilerParams(
            dimension_semantics=("arbitrary", "arbitrary"),
        ),
    )(gammas, params_t, base, ssr, dsr, fsr, sso, dso, fso, sro, dro, fro)


def kernel(base_scores, rel_ids, seen_sr, dt_sr, freq_sr, seen_so, dt_so,
           freq_so, seen_ro, dt_ro, freq_ro, lam_sr, wrec_sr, wfreq_sr,
           wstale_sr, bias_sr, lam_so, wrec_so, wfreq_so, bias_so, lam_ro,
           wrec_ro, wfreq_ro, bias_ro, gamma_exact, gamma_near):
    tables = (lam_sr, wrec_sr, wfreq_sr, wstale_sr, bias_sr,
              lam_so, wrec_so, wfreq_so, bias_so,
              lam_ro, wrec_ro, wfreq_ro, bias_ro)
    params_t = _sc_gather(tables, rel_ids.astype(jnp.int32)).T
    gammas = jnp.stack([gamma_exact, gamma_near]).astype(jnp.float32)

    tr = jnp.transpose
    logits_t, hb_t = _main(
        gammas, params_t, tr(base_scores),
        tr(seen_sr), tr(dt_sr), tr(freq_sr), tr(seen_so), tr(dt_so),
        tr(freq_so), tr(seen_ro), tr(dt_ro), tr(freq_ro))
    return (tr(logits_t), tr(hb_t))


# speedup vs baseline: 1.0658x; 1.0658x over previous
import functools

import jax
import jax.numpy as jnp
from jax import lax
from jax.experimental import pallas as pl
from jax.experimental.pallas import tpu as pltpu
from jax.experimental.pallas import tpu_sc as plsc

R = 1000
B = 1024
N = 10000
NPARAM = 128

def _sc_gather(tables, rel_ids):
    info = plsc.get_sparse_core_info()
    nc, ns = info.num_cores, info.num_subcores
    nw = nc * ns
    b_per_w = B // nw
    mesh = plsc.VectorSubcoreMesh(core_axis_name="c", subcore_axis_name="s")

    packed = jnp.pad(tables[0].T, ((0, NPARAM - 1), (0, 0)))
    for j in range(1, 13):
        packed = packed + jnp.pad(tables[j].T, ((j, NPARAM - 1 - j), (0, 0)))
    table = packed.T

    @functools.partial(
        pl.kernel,
        mesh=mesh,
        out_type=jax.ShapeDtypeStruct((B, NPARAM), jnp.float32),
        scratch_types=[
            pltpu.VMEM((b_per_w,), jnp.int32),
            pltpu.VMEM((b_per_w, NPARAM), jnp.float32),
            pltpu.SemaphoreType.DMA,
        ],
    )
    def k(table_hbm, idx_hbm, out_hbm, idx_v, rows_v, sem):
        wid = lax.axis_index("s") * nc + lax.axis_index("c")
        base = wid * b_per_w
        pltpu.sync_copy(idx_hbm.at[pl.ds(base, b_per_w)], idx_v)
        pltpu.async_copy(table_hbm.at[idx_v], rows_v, sem).wait()
        pltpu.sync_copy(rows_v, out_hbm.at[pl.ds(base, b_per_w)])

    return k(table, rel_ids)


CB = 256
CH = 1000


def _softplus(x):
    return jnp.maximum(x, 0.0) + jnp.log1p(jnp.exp(-jnp.abs(x)))


def _main_body(gam_ref, p_ref, base_ref,
               ssr_ref, dsr_ref, fsr_ref, sso_ref, dso_ref, fso_ref,
               sro_ref, dro_ref, fro_ref, logits_ref, hb_ref,
               csr_ref, cso_ref, cro_ref, msr_ref, mso_ref, mro_ref):
    p = pl.program_id(1)
    k = pl.program_id(2)

    @pl.when(p == 0)
    def _():
        for f_ref, c_ref, m_ref in ((fsr_ref, csr_ref, msr_ref),
                                    (fso_ref, cso_ref, mso_ref),
                                    (fro_ref, cro_ref, mro_ref)):
            x = f_ref[...]
            c_ref[pl.ds(k * CH, CH), :] = x
            mx = jnp.broadcast_to(jnp.max(x, axis=0, keepdims=True), (8, CB))

            @pl.when(k == 0)
            def _():
                m_ref[...] = mx

            @pl.when(k > 0)
            def _():
                m_ref[...] = jnp.maximum(m_ref[...], mx)

    @pl.when(p == 1)
    def _():
        def row(j):
            return p_ref[j:j + 1, :]

        def inv_norm(m_ref, wfreq):
            m = jnp.log1p(jnp.maximum(jnp.max(m_ref[...], axis=0,
                                              keepdims=True), 0.0))
            return wfreq / (m + 1e-8)

        def branch(seen, dt, c_ref, lam, wrec, inv, bias, wstale=None):
            freq = c_ref[pl.ds(k * CH, CH), :]
            dtf = jnp.log(1.0 + jnp.maximum(dt, 0.0))
            rec = jnp.exp(-lam * dtf) * seen
            ff = jnp.log(1.0 + jnp.maximum(freq, 0.0))
            score = wrec * rec + ff * inv * seen + bias
            if wstale is not None:
                score = score - wstale * (seen - rec * seen)
            return jnp.tanh(score) * seen

        g_sr = branch(ssr_ref[...], dsr_ref[...], csr_ref,
                      _softplus(row(0)), row(1), inv_norm(msr_ref, row(2)),
                      row(4), wstale=row(3))
        g_so = branch(sso_ref[...], dso_ref[...], cso_ref,
                      _softplus(row(5)), row(6), inv_norm(mso_ref, row(7)),
                      row(8))
        g_ro = branch(sro_ref[...], dro_ref[...], cro_ref,
                      _softplus(row(9)), row(10), inv_norm(mro_ref, row(11)),
                      row(12))

        hb = gam_ref[0] * g_sr + (gam_ref[1] * 0.5) * (g_so + g_ro)
        hb_ref[...] = hb
        logits_ref[...] = base_ref[...] + hb


def _main(gammas, params_t, base, ssr, dsr, fsr,
          sso, dso, fso, sro, dro, fro):
    other = pl.BlockSpec((CH, CB), lambda c, p, k: (k * p, c))
    freq = pl.BlockSpec((CH, CB), lambda c, p, k: (k * (1 - p), c))
    specs = [
        pl.BlockSpec(memory_space=pltpu.SMEM),
        pl.BlockSpec((NPARAM, CB), lambda c, p, k: (0, c)),
        other,
        other, other, freq,
        other, other, freq,
        other, other, freq,
    ]
    return pl.pallas_call(
        _main_body,
        grid=(B // CB, 2, N // CH),
        in_specs=specs,
        out_specs=[other, other],
        out_shape=[jax.ShapeDtypeStruct((N, B), jnp.float32)] * 2,
        scratch_shapes=[pltpu.VMEM((N, CB), jnp.float32)] * 3
        + [pltpu.VMEM((8, CB), jnp.float32)] * 3,
        compiler_params=pltpu.CompilerParams(
            dimension_semantics=("arbitrary", "arbitrary", "arbitrary"),
        ),
    )(gammas, params_t, base, ssr, dsr, fsr, sso, dso, fso, sro, dro, fro)


def kernel(base_scores, rel_ids, seen_sr, dt_sr, freq_sr, seen_so, dt_so,
           freq_so, seen_ro, dt_ro, freq_ro, lam_sr, wrec_sr, wfreq_sr,
           wstale_sr, bias_sr, lam_so, wrec_so, wfreq_so, bias_so, lam_ro,
           wrec_ro, wfreq_ro, bias_ro, gamma_exact, gamma_near):
    tables = (lam_sr, wrec_sr, wfreq_sr, wstale_sr, bias_sr,
              lam_so, wrec_so, wfreq_so, bias_so,
              lam_ro, wrec_ro, wfreq_ro, bias_ro)
    params_t = _sc_gather(tables, rel_ids.astype(jnp.int32)).T
    gammas = jnp.stack([gamma_exact, gamma_near]).astype(jnp.float32)

    tr = jnp.transpose
    logits_t, hb_t = _main(
        gammas, params_t, tr(base_scores),
        tr(seen_sr), tr(dt_sr), tr(freq_sr), tr(seen_so), tr(dt_so),
        tr(freq_so), tr(seen_ro), tr(dt_ro), tr(freq_ro))
    return (tr(logits_t), tr(hb_t))
